# baseline (speedup 1.0000x reference)
import functools

import jax
import jax.numpy as jnp
from jax.experimental import pallas as pl
from jax.experimental.pallas import tpu as pltpu

_TOP_K = 8
_BLK = 512


def _gate_body(x_ref, w_ref, idx_ref, wgt_ref):
    x = x_ref[...]
    w = w_ref[...]
    logits = jax.lax.dot_general(
        x, w, (((1,), (1,)), ((), ())),
        preferred_element_type=jnp.float32)

    col = jax.lax.broadcasted_iota(jnp.int32, logits.shape, 1)
    cur = logits
    vals = []
    idxs = []
    for _ in range(_TOP_K):
        m = jnp.max(cur, axis=1, keepdims=True)
        is_max = cur == m
        idx = jnp.min(jnp.where(is_max, col, logits.shape[1]),
                      axis=1, keepdims=True)
        vals.append(m)
        idxs.append(idx)
        cur = jnp.where(col == idx, -jnp.inf, cur)
    topv = jnp.concatenate(vals, axis=1)
    topi = jnp.concatenate(idxs, axis=1)

    e = jnp.exp(topv - topv[:, :1])
    wgt = e / jnp.sum(e, axis=1, keepdims=True)
    idx_ref[...] = topi
    wgt_ref[...] = wgt


@functools.partial(jax.jit, static_argnames=("interpret",))
def _gate(x, weight, interpret=False):
    tokens = x.shape[0]
    grid = (tokens // _BLK,)
    return pl.pallas_call(
        _gate_body,
        grid=grid,
        in_specs=[
            pl.BlockSpec((_BLK, x.shape[1]), lambda i: (i, 0)),
            pl.BlockSpec(weight.shape, lambda i: (0, 0)),
        ],
        out_specs=[
            pl.BlockSpec((_BLK, _TOP_K), lambda i: (i, 0)),
            pl.BlockSpec((_BLK, _TOP_K), lambda i: (i, 0)),
        ],
        out_shape=[
            jax.ShapeDtypeStruct((tokens, _TOP_K), jnp.int32),
            jax.ShapeDtypeStruct((tokens, _TOP_K), jnp.float32),
        ],
        interpret=interpret,
    )(x, weight)


def kernel(hidden_states, weight, interpret=False):
    bsz, seq_len, h = hidden_states.shape
    x = hidden_states.reshape(-1, h)
    topk_idx, topk_weight = _gate(x, weight, interpret=interpret)
    return (topk_idx, topk_weight, jnp.float32(0.0))


# speedup vs baseline: 1.7260x; 1.7260x over previous
import functools

import jax
import jax.numpy as jnp
from jax.experimental import pallas as pl
from jax.experimental.pallas import tpu as pltpu

_TOP_K = 8
_BLK = 512


def _gate_body(x_ref, w_ref, idx_ref, wgt_ref):
    x = x_ref[...]
    w = w_ref[...]
    logits = jax.lax.dot_general(
        w, x, (((1,), (1,)), ((), ())),
        preferred_element_type=jnp.float32)

    n_experts = logits.shape[0]
    row = jax.lax.broadcasted_iota(jnp.int32, logits.shape, 0)
    cur = logits
    vals = []
    idxs = []
    for k in range(_TOP_K):
        m = jnp.max(cur, axis=0, keepdims=True)
        is_max = cur == m
        idx = jnp.min(jnp.where(is_max, row, n_experts),
                      axis=0, keepdims=True)
        vals.append(m)
        idxs.append(idx)
        if k + 1 < _TOP_K:
            cur = jnp.where(row == idx, -jnp.inf, cur)
    topv = jnp.concatenate(vals, axis=0)
    topi = jnp.concatenate(idxs, axis=0)

    e = jnp.exp(topv - topv[:1, :])
    wgt = e / jnp.sum(e, axis=0, keepdims=True)
    idx_ref[...] = topi.T
    wgt_ref[...] = wgt.T


@functools.partial(jax.jit, static_argnames=("interpret",))
def _gate(x, weight, interpret=False):
    tokens = x.shape[0]
    grid = (tokens // _BLK,)
    return pl.pallas_call(
        _gate_body,
        grid=grid,
        in_specs=[
            pl.BlockSpec((_BLK, x.shape[1]), lambda i: (i, 0)),
            pl.BlockSpec(weight.shape, lambda i: (0, 0)),
        ],
        out_specs=[
            pl.BlockSpec((_BLK, _TOP_K), lambda i: (i, 0)),
            pl.BlockSpec((_BLK, _TOP_K), lambda i: (i, 0)),
        ],
        out_shape=[
            jax.ShapeDtypeStruct((tokens, _TOP_K), jnp.int32),
            jax.ShapeDtypeStruct((tokens, _TOP_K), jnp.float32),
        ],
        interpret=interpret,
    )(x, weight)


def kernel(hidden_states, weight, interpret=False):
    bsz, seq_len, h = hidden_states.shape
    x = hidden_states.reshape(-1, h)
    topk_idx, topk_weight = _gate(x, weight, interpret=interpret)
    return (topk_idx, topk_weight, jnp.float32(0.0))


# speedup vs baseline: 1.7321x; 1.0035x over previous
import functools

import jax
import jax.numpy as jnp
from jax.experimental import pallas as pl
from jax.experimental.pallas import tpu as pltpu

_TOP_K = 8
_BLK = 512


def _gate_body(x_ref, w_ref, idx_ref, wgt_ref):
    x = x_ref[...]
    w = w_ref[...]
    logits = jax.lax.dot_general(
        w, x, (((1,), (1,)), ((), ())),
        preferred_element_type=jnp.float32)

    n_experts = logits.shape[0]
    row = jax.lax.broadcasted_iota(jnp.int32, logits.shape, 0)
    cur = logits
    vals = []
    idxs = []
    for k in range(_TOP_K):
        m = jnp.max(cur, axis=0, keepdims=True)
        is_max = cur == m
        idx = jnp.min(jnp.where(is_max, row, n_experts),
                      axis=0, keepdims=True)
        vals.append(m)
        idxs.append(idx)
        if k + 1 < _TOP_K:
            cur = jnp.where(row == idx, -jnp.inf, cur)
    topv = jnp.concatenate(vals, axis=0)
    topi = jnp.concatenate(idxs, axis=0)

    e = jnp.exp(topv - topv[:1, :])
    wgt = e / jnp.sum(e, axis=0, keepdims=True)
    idx_ref[...] = topi.T
    wgt_ref[...] = wgt.T


@functools.partial(jax.jit, static_argnames=("interpret",))
def _gate(x, weight, interpret=False):
    tokens = x.shape[0]
    grid = (tokens // _BLK,)
    return pl.pallas_call(
        _gate_body,
        grid=grid,
        in_specs=[
            pl.BlockSpec((_BLK, x.shape[1]), lambda i: (i, 0)),
            pl.BlockSpec(weight.shape, lambda i: (0, 0)),
        ],
        out_specs=[
            pl.BlockSpec((_BLK, _TOP_K), lambda i: (i, 0)),
            pl.BlockSpec((_BLK, _TOP_K), lambda i: (i, 0)),
        ],
        out_shape=[
            jax.ShapeDtypeStruct((tokens, _TOP_K), jnp.int32),
            jax.ShapeDtypeStruct((tokens, _TOP_K), jnp.float32),
        ],
        compiler_params=pltpu.CompilerParams(
            dimension_semantics=("parallel",)),
        interpret=interpret,
    )(x, weight)


def kernel(hidden_states, weight, interpret=False):
    bsz, seq_len, h = hidden_states.shape
    x = hidden_states.reshape(-1, h)
    topk_idx, topk_weight = _gate(x, weight, interpret=interpret)
    return (topk_idx, topk_weight, jnp.float32(0.0))


# speedup vs baseline: 2.0114x; 1.1613x over previous
import functools

import jax
import jax.numpy as jnp
from jax.experimental import pallas as pl
from jax.experimental.pallas import tpu as pltpu

_TOP_K = 8
_BLK = 1024


def _gate_body(x_ref, w_ref, idx_ref, wgt_ref):
    x = x_ref[...]
    w = w_ref[...]
    logits = jax.lax.dot_general(
        w, x, (((1,), (1,)), ((), ())),
        preferred_element_type=jnp.float32)

    n_experts = logits.shape[0]
    row = jax.lax.broadcasted_iota(jnp.int32, logits.shape, 0)
    cur = logits
    vals = []
    idxs = []
    for k in range(_TOP_K):
        m = jnp.max(cur, axis=0, keepdims=True)
        is_max = cur == m
        idx = jnp.min(jnp.where(is_max, row, n_experts),
                      axis=0, keepdims=True)
        vals.append(m)
        idxs.append(idx)
        if k + 1 < _TOP_K:
            cur = jnp.where(row == idx, -jnp.inf, cur)
    topv = jnp.concatenate(vals, axis=0)
    topi = jnp.concatenate(idxs, axis=0)

    e = jnp.exp(topv - topv[:1, :])
    wgt = e / jnp.sum(e, axis=0, keepdims=True)
    idx_ref[...] = topi.T
    wgt_ref[...] = wgt.T


@functools.partial(jax.jit, static_argnames=("interpret",))
def _gate(x, weight, interpret=False):
    tokens = x.shape[0]
    grid = (tokens // _BLK,)
    return pl.pallas_call(
        _gate_body,
        grid=grid,
        in_specs=[
            pl.BlockSpec((_BLK, x.shape[1]), lambda i: (i, 0)),
            pl.BlockSpec(weight.shape, lambda i: (0, 0)),
        ],
        out_specs=[
            pl.BlockSpec((_BLK, _TOP_K), lambda i: (i, 0)),
            pl.BlockSpec((_BLK, _TOP_K), lambda i: (i, 0)),
        ],
        out_shape=[
            jax.ShapeDtypeStruct((tokens, _TOP_K), jnp.int32),
            jax.ShapeDtypeStruct((tokens, _TOP_K), jnp.float32),
        ],
        compiler_params=pltpu.CompilerParams(
            dimension_semantics=("parallel",)),
        interpret=interpret,
    )(x, weight)


def kernel(hidden_states, weight, interpret=False):
    bsz, seq_len, h = hidden_states.shape
    x = hidden_states.reshape(-1, h)
    topk_idx, topk_weight = _gate(x, weight, interpret=interpret)
    return (topk_idx, topk_weight, jnp.float32(0.0))


# speedup vs baseline: 2.1086x; 1.0483x over previous
import functools

import jax
import jax.numpy as jnp
from jax.experimental import pallas as pl
from jax.experimental.pallas import tpu as pltpu

_TOP_K = 8
_BLK = 2048


def _gate_body(x_ref, w_ref, idx_ref, wgt_ref):
    x = x_ref[...]
    w = w_ref[...]
    logits = jax.lax.dot_general(
        w, x, (((1,), (1,)), ((), ())),
        preferred_element_type=jnp.float32)

    n_experts = logits.shape[0]
    row = jax.lax.broadcasted_iota(jnp.int32, logits.shape, 0)
    cur = logits
    vals = []
    idxs = []
    for k in range(_TOP_K):
        m = jnp.max(cur, axis=0, keepdims=True)
        is_max = cur == m
        idx = jnp.min(jnp.where(is_max, row, n_experts),
                      axis=0, keepdims=True)
        vals.append(m)
        idxs.append(idx)
        if k + 1 < _TOP_K:
            cur = jnp.where(row == idx, -jnp.inf, cur)
    topv = jnp.concatenate(vals, axis=0)
    topi = jnp.concatenate(idxs, axis=0)

    e = jnp.exp(topv - topv[:1, :])
    wgt = e / jnp.sum(e, axis=0, keepdims=True)
    idx_ref[...] = topi.T
    wgt_ref[...] = wgt.T


@functools.partial(jax.jit, static_argnames=("interpret",))
def _gate(x, weight, interpret=False):
    tokens = x.shape[0]
    grid = (tokens // _BLK,)
    return pl.pallas_call(
        _gate_body,
        grid=grid,
        in_specs=[
            pl.BlockSpec((_BLK, x.shape[1]), lambda i: (i, 0)),
            pl.BlockSpec(weight.shape, lambda i: (0, 0)),
        ],
        out_specs=[
            pl.BlockSpec((_BLK, _TOP_K), lambda i: (i, 0)),
            pl.BlockSpec((_BLK, _TOP_K), lambda i: (i, 0)),
        ],
        out_shape=[
            jax.ShapeDtypeStruct((tokens, _TOP_K), jnp.int32),
            jax.ShapeDtypeStruct((tokens, _TOP_K), jnp.float32),
        ],
        compiler_params=pltpu.CompilerParams(
            dimension_semantics=("parallel",)),
        interpret=interpret,
    )(x, weight)


def kernel(hidden_states, weight, interpret=False):
    bsz, seq_len, h = hidden_states.shape
    x = hidden_states.reshape(-1, h)
    topk_idx, topk_weight = _gate(x, weight, interpret=interpret)
    return (topk_idx, topk_weight, jnp.float32(0.0))


# speedup vs baseline: 2.1091x; 1.0002x over previous
import functools

import jax
import jax.numpy as jnp
from jax.experimental import pallas as pl
from jax.experimental.pallas import tpu as pltpu

_TOP_K = 8
_BLK = 2048


def _gate_body(x1_ref, x2_ref, w_ref, idx_ref, wgt_ref):
    x1 = x1_ref[...]
    x2 = x2_ref[...]
    w = w_ref[...]
    h2 = x1.shape[1]
    dn = (((1,), (1,)), ((), ()))
    logits = jax.lax.dot_general(
        w[:, :h2], x1, dn, preferred_element_type=jnp.float32)
    logits = logits + jax.lax.dot_general(
        w[:, h2:], x2, dn, preferred_element_type=jnp.float32)

    n_experts = logits.shape[0]
    row = jax.lax.broadcasted_iota(jnp.int32, logits.shape, 0)
    cur = logits
    vals = []
    idxs = []
    for k in range(_TOP_K):
        m = jnp.max(cur, axis=0, keepdims=True)
        is_max = cur == m
        idx = jnp.min(jnp.where(is_max, row, n_experts),
                      axis=0, keepdims=True)
        vals.append(m)
        idxs.append(idx)
        if k + 1 < _TOP_K:
            cur = jnp.where(row == idx, -jnp.inf, cur)
    topv = jnp.concatenate(vals, axis=0)
    topi = jnp.concatenate(idxs, axis=0)

    e = jnp.exp(topv - topv[:1, :])
    wgt = e / jnp.sum(e, axis=0, keepdims=True)
    idx_ref[...] = topi.T
    wgt_ref[...] = wgt.T


@functools.partial(jax.jit, static_argnames=("interpret",))
def _gate(x, weight, interpret=False):
    tokens = x.shape[0]
    grid = (tokens // _BLK,)
    return pl.pallas_call(
        _gate_body,
        grid=grid,
        in_specs=[
            pl.BlockSpec((_BLK, x.shape[1] // 2), lambda i: (i, 0)),
            pl.BlockSpec((_BLK, x.shape[1] // 2), lambda i: (i, 1)),
            pl.BlockSpec(weight.shape, lambda i: (0, 0)),
        ],
        out_specs=[
            pl.BlockSpec((_BLK, _TOP_K), lambda i: (i, 0)),
            pl.BlockSpec((_BLK, _TOP_K), lambda i: (i, 0)),
        ],
        out_shape=[
            jax.ShapeDtypeStruct((tokens, _TOP_K), jnp.int32),
            jax.ShapeDtypeStruct((tokens, _TOP_K), jnp.float32),
        ],
        compiler_params=pltpu.CompilerParams(
            dimension_semantics=("parallel",)),
        interpret=interpret,
    )(x, x, weight)


def kernel(hidden_states, weight, interpret=False):
    bsz, seq_len, h = hidden_states.shape
    x = hidden_states.reshape(-1, h)
    topk_idx, topk_weight = _gate(x, weight, interpret=interpret)
    return (topk_idx, topk_weight, jnp.float32(0.0))
